# baseline (speedup 1.0000x reference)
import functools

import jax
import jax.numpy as jnp
from jax import lax
from jax.experimental import pallas as pl
from jax.experimental.pallas import tpu as pltpu
from jax.experimental.pallas import tpu_sc as plsc

N_PAD = 10240
C_PAD = 40
NC, NS = 2, 16
NW = NC * NS
EB = 128
KG = 8
ROWS_W = N_PAD // NW
ROWS_S = N_PAD // NS


def _sc_mesh():
    return plsc.VectorSubcoreMesh(core_axis_name="c", subcore_axis_name="s")


def _make_deg_kernel(nb):
    @functools.partial(
        pl.kernel,
        mesh=_sc_mesh(),
        out_type=jax.ShapeDtypeStruct((NW, N_PAD), jnp.float32),
        scratch_types=[
            pltpu.VMEM((nb, EB), jnp.int32),
            pltpu.VMEM((N_PAD,), jnp.float32),
        ],
        compiler_params=pltpu.CompilerParams(needs_layout_passes=False),
    )
    def deg_kernel(dst_hbm, zeros_hbm, out_hbm, dst_v, deg_v):
        c = lax.axis_index("c")
        s = lax.axis_index("s")
        wid = c * NS + s

        pltpu.sync_copy(zeros_hbm, deg_v)
        pltpu.sync_copy(dst_hbm.at[wid], dst_v)

        ones = jnp.ones((16,), jnp.float32)

        def edge_body(b, _):
            for j in range(EB // 16):
                idx = dst_v[b, pl.ds(j * 16, 16)]
                plsc.addupdate_scatter(deg_v, [idx], ones)
            return 0

        lax.fori_loop(0, nb, edge_body, 0)

        pltpu.sync_copy(deg_v, out_hbm.at[wid])

    return deg_kernel


def _make_hop_kernel(nb):
    @functools.partial(
        pl.kernel,
        mesh=_sc_mesh(),
        out_type=jax.ShapeDtypeStruct((NC * N_PAD, C_PAD), jnp.float32),
        scratch_types=[
            pltpu.VMEM((nb, EB), jnp.int32),
            pltpu.VMEM((nb, EB), jnp.int32),
            pltpu.VMEM((EB, C_PAD), jnp.float32),
            pltpu.VMEM((EB, C_PAD), jnp.float32),
            pltpu.VMEM_SHARED((N_PAD, C_PAD), jnp.float32),
            pltpu.VMEM_SHARED((N_PAD, C_PAD), jnp.float32),
            pltpu.SemaphoreType.DMA,
            pltpu.SemaphoreType.DMA,
        ],
        compiler_params=pltpu.CompilerParams(use_tc_tiling_on_sc=False),
    )
    def hop_kernel(z_hbm, src_hbm, dst_hbm, out_hbm, src_v, dst_v, buf_a, buf_b,
                   z_s, acc, sem_a, sem_b):
        c = lax.axis_index("c")
        s = lax.axis_index("s")
        wid = c * NS + s

        r0 = s * ROWS_S
        pltpu.sync_copy(z_hbm.at[pl.ds(r0, ROWS_S)], z_s.at[pl.ds(r0, ROWS_S)])
        pltpu.sync_copy(z_hbm.at[pl.ds(r0, ROWS_S)], acc.at[pl.ds(r0, ROWS_S)])
        pltpu.sync_copy(src_hbm.at[wid], src_v)
        pltpu.sync_copy(dst_hbm.at[wid], dst_v)
        plsc.subcore_barrier()

        def gather(b, buf, sem):
            pltpu.async_copy(z_s.at[src_v.at[b]], buf, sem)

        def gwait(buf, sem):
            pltpu.make_async_copy(z_hbm.at[pl.ds(0, EB)], buf, sem).wait()

        def scatter(b, buf):
            pltpu.sync_copy(buf, acc.at[dst_v.at[b]], add=True)

        gather(0, buf_a, sem_a)

        def edge_body(i, _):
            b0 = 2 * i
            gather(b0 + 1, buf_b, sem_b)
            gwait(buf_a, sem_a)
            scatter(b0, buf_a)
            gather(b0 + 2, buf_a, sem_a)
            gwait(buf_b, sem_b)
            scatter(b0 + 1, buf_b)
            return 0

        lax.fori_loop(0, nb // 2 - 1, edge_body, 0)

        gather(nb - 1, buf_b, sem_b)
        gwait(buf_a, sem_a)
        scatter(nb - 2, buf_a)
        gwait(buf_b, sem_b)
        scatter(nb - 1, buf_b)

        plsc.subcore_barrier()
        pltpu.sync_copy(
            acc.at[pl.ds(r0, ROWS_S)],
            out_hbm.at[pl.ds(c * N_PAD + r0, ROWS_S)],
        )

    return hop_kernel


_RB = 1024


def _tc_a_body(x_ref, w_ref, degp_ref, z1_ref, norm_ref, invd_ref):
    deg = 1.0 + jnp.sum(degp_ref[...], axis=0)
    norm = lax.rsqrt(deg)
    invd = 1.0 / deg
    y = jnp.dot(x_ref[...], w_ref[...], preferred_element_type=jnp.float32)
    z1_ref[...] = y * norm[:, None]
    norm_ref[...] = norm[:, None]
    invd_ref[...] = invd[:, None]


def _tc_a(x_pad, w_pad, deg_parts):
    grid = (N_PAD // _RB,)
    return pl.pallas_call(
        _tc_a_body,
        grid=grid,
        in_specs=[
            pl.BlockSpec((_RB, 128), lambda i: (i, 0)),
            pl.BlockSpec((128, C_PAD), lambda i: (0, 0)),
            pl.BlockSpec((NW, _RB), lambda i: (0, i)),
        ],
        out_specs=[
            pl.BlockSpec((_RB, C_PAD), lambda i: (i, 0)),
            pl.BlockSpec((_RB, 1), lambda i: (i, 0)),
            pl.BlockSpec((_RB, 1), lambda i: (i, 0)),
        ],
        out_shape=[
            jax.ShapeDtypeStruct((N_PAD, C_PAD), jnp.float32),
            jax.ShapeDtypeStruct((N_PAD, 1), jnp.float32),
            jax.ShapeDtypeStruct((N_PAD, 1), jnp.float32),
        ],
    )(x_pad, w_pad, deg_parts)


def _tc_combine_body(p_ref, prev_ref, scale_ref, out_ref):
    tot = p_ref[0] + p_ref[1] - prev_ref[...]
    out_ref[...] = tot * scale_ref[...]


def _tc_combine(parts, prev, scale):
    grid = (N_PAD // _RB,)
    return pl.pallas_call(
        _tc_combine_body,
        grid=grid,
        in_specs=[
            pl.BlockSpec((2, _RB, C_PAD), lambda i: (0, i, 0)),
            pl.BlockSpec((_RB, C_PAD), lambda i: (i, 0)),
            pl.BlockSpec((_RB, 1), lambda i: (i, 0)),
        ],
        out_specs=pl.BlockSpec((_RB, C_PAD), lambda i: (i, 0)),
        out_shape=jax.ShapeDtypeStruct((N_PAD, C_PAD), jnp.float32),
    )(parts, prev, scale)


def kernel(x, edge_index, W):
    n, d = x.shape
    c_out = W.shape[1]
    e = edge_index.shape[1]

    epw = -(-e // NW)
    epw = -(-epw // (KG * EB)) * (KG * EB)
    nb = epw // EB
    e_pad = NW * epw
    src = edge_index[0].astype(jnp.int32)
    dst = edge_index[1].astype(jnp.int32)
    fill = jnp.full((e_pad - e,), N_PAD - 1, jnp.int32)
    src3 = jnp.concatenate([src, fill]).reshape(NW, nb, EB)
    dst3 = jnp.concatenate([dst, fill]).reshape(NW, nb, EB)

    x_pad = jnp.zeros((N_PAD, d), x.dtype).at[:n].set(x)
    w_pad = jnp.zeros((d, C_PAD), W.dtype).at[:, :c_out].set(W)

    deg_parts = _make_deg_kernel(nb)(dst3, jnp.zeros((N_PAD,), jnp.float32))
    z1, norm, invd = _tc_a(x_pad, w_pad, deg_parts)

    hop = _make_hop_kernel(nb)
    p1 = hop(z1, src3, dst3).reshape(NC, N_PAD, C_PAD)
    z3 = _tc_combine(p1, z1, invd)
    p2 = hop(z3, src3, dst3).reshape(NC, N_PAD, C_PAD)
    out = _tc_combine(p2, z3, norm)

    return out[:n, :c_out]


# speedup vs baseline: 1.0156x; 1.0156x over previous
import functools

import jax
import jax.numpy as jnp
from jax import lax
from jax.experimental import pallas as pl
from jax.experimental.pallas import tpu as pltpu
from jax.experimental.pallas import tpu_sc as plsc

N_PAD = 10240
C_PAD = 40
NC, NS = 2, 16
NW = NC * NS
EB = 128
KG = 8
ROWS_W = N_PAD // NW
ROWS_S = N_PAD // NS


def _sc_mesh():
    return plsc.VectorSubcoreMesh(core_axis_name="c", subcore_axis_name="s")


def _make_deg_kernel(nb):
    @functools.partial(
        pl.kernel,
        mesh=_sc_mesh(),
        out_type=jax.ShapeDtypeStruct((NW, N_PAD), jnp.float32),
        scratch_types=[
            pltpu.VMEM((nb, EB), jnp.int32),
            pltpu.VMEM((N_PAD,), jnp.float32),
        ],
        compiler_params=pltpu.CompilerParams(needs_layout_passes=False),
    )
    def deg_kernel(dst_hbm, zeros_hbm, out_hbm, dst_v, deg_v):
        c = lax.axis_index("c")
        s = lax.axis_index("s")
        wid = c * NS + s

        pltpu.sync_copy(zeros_hbm, deg_v)
        pltpu.sync_copy(dst_hbm.at[wid], dst_v)

        ones = jnp.ones((16,), jnp.float32)

        def edge_body(b, _):
            for j in range(EB // 16):
                idx = dst_v[b, pl.ds(j * 16, 16)]
                plsc.addupdate_scatter(deg_v, [idx], ones)
            return 0

        lax.fori_loop(0, nb, edge_body, 0)

        pltpu.sync_copy(deg_v, out_hbm.at[wid])

    return deg_kernel


def _make_hop_kernel(nb):
    @functools.partial(
        pl.kernel,
        mesh=_sc_mesh(),
        out_type=jax.ShapeDtypeStruct((NC * N_PAD, C_PAD), jnp.float32),
        scratch_types=[
            pltpu.VMEM((nb, EB), jnp.int32),
            pltpu.VMEM((nb, EB), jnp.int32),
            pltpu.VMEM((EB, C_PAD), jnp.float32),
            pltpu.VMEM((EB, C_PAD), jnp.float32),
            pltpu.VMEM_SHARED((N_PAD, C_PAD), jnp.float32),
            pltpu.VMEM_SHARED((N_PAD, C_PAD), jnp.float32),
            pltpu.SemaphoreType.DMA,
            pltpu.SemaphoreType.DMA,
        ],
        compiler_params=pltpu.CompilerParams(use_tc_tiling_on_sc=False),
    )
    def hop_kernel(z_hbm, src_hbm, dst_hbm, out_hbm, src_v, dst_v, buf_a, buf_b,
                   z_s, acc, sem_a, sem_b):
        c = lax.axis_index("c")
        s = lax.axis_index("s")
        wid = c * NS + s

        r0 = s * ROWS_S
        stage = [
            pltpu.async_copy(
                z_hbm.at[pl.ds(r0, ROWS_S)], z_s.at[pl.ds(r0, ROWS_S)], sem_a
            ),
            pltpu.async_copy(
                z_hbm.at[pl.ds(r0, ROWS_S)], acc.at[pl.ds(r0, ROWS_S)], sem_a
            ),
            pltpu.async_copy(src_hbm.at[wid], src_v, sem_b),
            pltpu.async_copy(dst_hbm.at[wid], dst_v, sem_b),
        ]
        for d in stage:
            d.wait()
        plsc.subcore_barrier()

        def gather(b, buf, sem):
            pltpu.async_copy(z_s.at[src_v.at[b]], buf, sem)

        def gwait(buf, sem):
            pltpu.make_async_copy(z_hbm.at[pl.ds(0, EB)], buf, sem).wait()

        def scatter(b, buf):
            pltpu.sync_copy(buf, acc.at[dst_v.at[b]], add=True)

        gather(0, buf_a, sem_a)

        def edge_body(i, _):
            b0 = 2 * i
            gather(b0 + 1, buf_b, sem_b)
            gwait(buf_a, sem_a)
            scatter(b0, buf_a)
            gather(b0 + 2, buf_a, sem_a)
            gwait(buf_b, sem_b)
            scatter(b0 + 1, buf_b)
            return 0

        lax.fori_loop(0, nb // 2 - 1, edge_body, 0)

        gather(nb - 1, buf_b, sem_b)
        gwait(buf_a, sem_a)
        scatter(nb - 2, buf_a)
        gwait(buf_b, sem_b)
        scatter(nb - 1, buf_b)

        plsc.subcore_barrier()
        pltpu.sync_copy(
            acc.at[pl.ds(r0, ROWS_S)],
            out_hbm.at[pl.ds(c * N_PAD + r0, ROWS_S)],
        )

    return hop_kernel


_RB = 1024


def _tc_a_body(x_ref, w_ref, degp_ref, z1_ref, norm_ref, invd_ref):
    deg = 1.0 + jnp.sum(degp_ref[...], axis=0)
    norm = lax.rsqrt(deg)
    invd = 1.0 / deg
    y = jnp.dot(x_ref[...], w_ref[...], preferred_element_type=jnp.float32)
    z1_ref[...] = y * norm[:, None]
    norm_ref[...] = norm[:, None]
    invd_ref[...] = invd[:, None]


def _tc_a(x_pad, w_pad, deg_parts):
    grid = (N_PAD // _RB,)
    return pl.pallas_call(
        _tc_a_body,
        grid=grid,
        in_specs=[
            pl.BlockSpec((_RB, 128), lambda i: (i, 0)),
            pl.BlockSpec((128, C_PAD), lambda i: (0, 0)),
            pl.BlockSpec((NW, _RB), lambda i: (0, i)),
        ],
        out_specs=[
            pl.BlockSpec((_RB, C_PAD), lambda i: (i, 0)),
            pl.BlockSpec((_RB, 1), lambda i: (i, 0)),
            pl.BlockSpec((_RB, 1), lambda i: (i, 0)),
        ],
        out_shape=[
            jax.ShapeDtypeStruct((N_PAD, C_PAD), jnp.float32),
            jax.ShapeDtypeStruct((N_PAD, 1), jnp.float32),
            jax.ShapeDtypeStruct((N_PAD, 1), jnp.float32),
        ],
    )(x_pad, w_pad, deg_parts)


def _tc_combine_body(p_ref, prev_ref, scale_ref, out_ref):
    tot = p_ref[0] + p_ref[1] - prev_ref[...]
    out_ref[...] = tot * scale_ref[...]


def _tc_combine(parts, prev, scale):
    grid = (N_PAD // _RB,)
    return pl.pallas_call(
        _tc_combine_body,
        grid=grid,
        in_specs=[
            pl.BlockSpec((2, _RB, C_PAD), lambda i: (0, i, 0)),
            pl.BlockSpec((_RB, C_PAD), lambda i: (i, 0)),
            pl.BlockSpec((_RB, 1), lambda i: (i, 0)),
        ],
        out_specs=pl.BlockSpec((_RB, C_PAD), lambda i: (i, 0)),
        out_shape=jax.ShapeDtypeStruct((N_PAD, C_PAD), jnp.float32),
    )(parts, prev, scale)


def kernel(x, edge_index, W):
    n, d = x.shape
    c_out = W.shape[1]
    e = edge_index.shape[1]

    epw = -(-e // NW)
    epw = -(-epw // (KG * EB)) * (KG * EB)
    nb = epw // EB
    e_pad = NW * epw
    src = edge_index[0].astype(jnp.int32)
    dst = edge_index[1].astype(jnp.int32)
    fill = jnp.full((e_pad - e,), N_PAD - 1, jnp.int32)
    src3 = jnp.concatenate([src, fill]).reshape(NW, nb, EB)
    dst3 = jnp.concatenate([dst, fill]).reshape(NW, nb, EB)

    x_pad = jnp.zeros((N_PAD, d), x.dtype).at[:n].set(x)
    w_pad = jnp.zeros((d, C_PAD), W.dtype).at[:, :c_out].set(W)

    deg_parts = _make_deg_kernel(nb)(dst3, jnp.zeros((N_PAD,), jnp.float32))
    z1, norm, invd = _tc_a(x_pad, w_pad, deg_parts)

    hop = _make_hop_kernel(nb)
    p1 = hop(z1, src3, dst3).reshape(NC, N_PAD, C_PAD)
    z3 = _tc_combine(p1, z1, invd)
    p2 = hop(z3, src3, dst3).reshape(NC, N_PAD, C_PAD)
    out = _tc_combine(p2, z3, norm)

    return out[:n, :c_out]


# speedup vs baseline: 1.0176x; 1.0020x over previous
import functools

import jax
import jax.numpy as jnp
from jax import lax
from jax.experimental import pallas as pl
from jax.experimental.pallas import tpu as pltpu
from jax.experimental.pallas import tpu_sc as plsc

N_PAD = 10240
C_PAD = 40
NC, NS = 2, 16
NW = NC * NS
EB = 128
PB = 8
ROWS_W = N_PAD // NW
ROWS_S = N_PAD // NS


def _sc_mesh():
    return plsc.VectorSubcoreMesh(core_axis_name="c", subcore_axis_name="s")


def _make_deg_kernel(nb):
    @functools.partial(
        pl.kernel,
        mesh=_sc_mesh(),
        out_type=jax.ShapeDtypeStruct((NW, N_PAD), jnp.float32),
        scratch_types=[
            pltpu.VMEM((nb, EB), jnp.int32),
            pltpu.VMEM((N_PAD,), jnp.float32),
        ],
        compiler_params=pltpu.CompilerParams(needs_layout_passes=False),
    )
    def deg_kernel(dst_hbm, zeros_hbm, out_hbm, dst_v, deg_v):
        c = lax.axis_index("c")
        s = lax.axis_index("s")
        wid = c * NS + s

        pltpu.sync_copy(zeros_hbm, deg_v)
        pltpu.sync_copy(dst_hbm.at[wid], dst_v)

        ones = jnp.ones((16,), jnp.float32)

        def edge_body(b, _):
            for j in range(EB // 16):
                idx = dst_v[b, pl.ds(j * 16, 16)]
                plsc.addupdate_scatter(deg_v, [idx], ones)
            return 0

        lax.fori_loop(0, nb, edge_body, 0)

        pltpu.sync_copy(deg_v, out_hbm.at[wid])

    return deg_kernel


def _make_hop_kernel(nb):
    @functools.partial(
        pl.kernel,
        mesh=_sc_mesh(),
        out_type=jax.ShapeDtypeStruct((NC * N_PAD, C_PAD), jnp.float32),
        scratch_types=[
            pltpu.VMEM((nb, EB), jnp.int32),
            pltpu.VMEM((nb, EB), jnp.int32),
            pltpu.VMEM((EB, C_PAD), jnp.float32),
            pltpu.VMEM((EB, C_PAD), jnp.float32),
            pltpu.VMEM_SHARED((N_PAD, C_PAD), jnp.float32),
            pltpu.VMEM_SHARED((N_PAD, C_PAD), jnp.float32),
            pltpu.SemaphoreType.DMA,
            pltpu.SemaphoreType.DMA,
        ],
        compiler_params=pltpu.CompilerParams(use_tc_tiling_on_sc=False),
    )
    def hop_kernel(z_hbm, src_hbm, dst_hbm, out_hbm, src_v, dst_v, buf_a, buf_b,
                   z_s, acc, sem_a, sem_b):
        c = lax.axis_index("c")
        s = lax.axis_index("s")
        wid = c * NS + s

        r0 = s * ROWS_S
        stage = [
            pltpu.async_copy(
                z_hbm.at[pl.ds(r0, ROWS_S)], z_s.at[pl.ds(r0, ROWS_S)], sem_a
            ),
            pltpu.async_copy(
                z_hbm.at[pl.ds(r0, ROWS_S)], acc.at[pl.ds(r0, ROWS_S)], sem_a
            ),
            pltpu.async_copy(src_hbm.at[wid], src_v, sem_b),
            pltpu.async_copy(dst_hbm.at[wid], dst_v, sem_b),
        ]
        for d in stage:
            d.wait()
        plsc.subcore_barrier()

        def gather(b, buf, sem):
            pltpu.async_copy(z_s.at[src_v.at[b]], buf, sem)

        def gwait(buf, sem):
            pltpu.make_async_copy(z_hbm.at[pl.ds(0, EB)], buf, sem).wait()

        def scatter(b, buf):
            pltpu.sync_copy(buf, acc.at[dst_v.at[b]], add=True)

        gather(0, buf_a, sem_a)

        def edge_body(i, _):
            b0 = 2 * i
            gather(b0 + 1, buf_b, sem_b)
            gwait(buf_a, sem_a)
            scatter(b0, buf_a)
            gather(b0 + 2, buf_a, sem_a)
            gwait(buf_b, sem_b)
            scatter(b0 + 1, buf_b)
            return 0

        lax.fori_loop(0, nb // 2 - 1, edge_body, 0)

        gather(nb - 1, buf_b, sem_b)
        gwait(buf_a, sem_a)
        scatter(nb - 2, buf_a)
        gwait(buf_b, sem_b)
        scatter(nb - 1, buf_b)

        plsc.subcore_barrier()
        pltpu.sync_copy(
            acc.at[pl.ds(r0, ROWS_S)],
            out_hbm.at[pl.ds(c * N_PAD + r0, ROWS_S)],
        )

    return hop_kernel


_RB = 1024


def _tc_a_body(x_ref, w_ref, degp_ref, z1_ref, norm_ref, invd_ref):
    deg = 1.0 + jnp.sum(degp_ref[...], axis=0)
    norm = lax.rsqrt(deg)
    invd = 1.0 / deg
    y = jnp.dot(x_ref[...], w_ref[...], preferred_element_type=jnp.float32)
    z1_ref[...] = y * norm[:, None]
    norm_ref[...] = norm[:, None]
    invd_ref[...] = invd[:, None]


def _tc_a(x_pad, w_pad, deg_parts):
    grid = (N_PAD // _RB,)
    return pl.pallas_call(
        _tc_a_body,
        grid=grid,
        in_specs=[
            pl.BlockSpec((_RB, 128), lambda i: (i, 0)),
            pl.BlockSpec((128, C_PAD), lambda i: (0, 0)),
            pl.BlockSpec((NW, _RB), lambda i: (0, i)),
        ],
        out_specs=[
            pl.BlockSpec((_RB, C_PAD), lambda i: (i, 0)),
            pl.BlockSpec((_RB, 1), lambda i: (i, 0)),
            pl.BlockSpec((_RB, 1), lambda i: (i, 0)),
        ],
        out_shape=[
            jax.ShapeDtypeStruct((N_PAD, C_PAD), jnp.float32),
            jax.ShapeDtypeStruct((N_PAD, 1), jnp.float32),
            jax.ShapeDtypeStruct((N_PAD, 1), jnp.float32),
        ],
    )(x_pad, w_pad, deg_parts)


def _tc_combine_body(p_ref, prev_ref, scale_ref, out_ref):
    tot = p_ref[0] + p_ref[1] - prev_ref[...]
    out_ref[...] = tot * scale_ref[...]


def _tc_combine(parts, prev, scale):
    grid = (N_PAD // _RB,)
    return pl.pallas_call(
        _tc_combine_body,
        grid=grid,
        in_specs=[
            pl.BlockSpec((2, _RB, C_PAD), lambda i: (0, i, 0)),
            pl.BlockSpec((_RB, C_PAD), lambda i: (i, 0)),
            pl.BlockSpec((_RB, 1), lambda i: (i, 0)),
        ],
        out_specs=pl.BlockSpec((_RB, C_PAD), lambda i: (i, 0)),
        out_shape=jax.ShapeDtypeStruct((N_PAD, C_PAD), jnp.float32),
    )(parts, prev, scale)


def kernel(x, edge_index, W):
    n, d = x.shape
    c_out = W.shape[1]
    e = edge_index.shape[1]

    epw = -(-e // NW)
    epw = -(-epw // (PB * EB)) * (PB * EB)
    nb = epw // EB
    e_pad = NW * epw
    src = edge_index[0].astype(jnp.int32)
    dst = edge_index[1].astype(jnp.int32)
    fill = jnp.full((e_pad - e,), N_PAD - 1, jnp.int32)
    src3 = jnp.concatenate([src, fill]).reshape(NW, nb, EB)
    dst3 = jnp.concatenate([dst, fill]).reshape(NW, nb, EB)

    x_pad = jnp.zeros((N_PAD, d), x.dtype).at[:n].set(x)
    w_pad = jnp.zeros((d, C_PAD), W.dtype).at[:, :c_out].set(W)

    deg_parts = _make_deg_kernel(nb)(dst3, jnp.zeros((N_PAD,), jnp.float32))
    z1, norm, invd = _tc_a(x_pad, w_pad, deg_parts)

    hop = _make_hop_kernel(nb)
    p1 = hop(z1, src3, dst3).reshape(NC, N_PAD, C_PAD)
    z3 = _tc_combine(p1, z1, invd)
    p2 = hop(z3, src3, dst3).reshape(NC, N_PAD, C_PAD)
    out = _tc_combine(p2, z3, norm)

    return out[:n, :c_out]
